# baseline (speedup 1.0000x reference)
import functools

import jax
import jax.numpy as jnp
from jax import lax
from jax.experimental import pallas as pl
from jax.experimental.pallas import tpu as pltpu
from jax.experimental.pallas import tpu_sc as plsc

_B = 4096
_C = 10000
_NC, _NS, _L = 2, 16, 16
_NW = _NC * _NS
_NPW = _B // _NW
_SR = 8
_NSTRIPE = _NPW // _SR
_CW = 1664
_NCC = 6
_U = 8


def _sc_body(cos_hbm, out_hbm, buf0, buf1, acc_v, sem0, sem1):
    wid = lax.axis_index("s") * _NC + lax.axis_index("c")
    row0 = wid * _NPW
    bufs = (buf0, buf1)
    sems = (sem0, sem1)

    def issue(s, cc, buf, sem):
        pltpu.async_copy(
            cos_hbm.at[pl.ds(row0 + s * _SR, _SR), pl.ds(cc * _CW, _CW)],
            buf, sem,
        )

    def wait(buf, sem):
        pltpu.make_async_copy(
            cos_hbm.at[pl.ds(row0, _SR), pl.ds(0, _CW)], buf, sem
        ).wait()

    def chunk_sum(buf, total):
        for rr in range(_SR):
            def inner(j, acc):
                off = j * (_U * _L)
                for u in range(_U):
                    acc = acc + buf[rr, pl.ds(off + u * _L, _L)]
                return acc

            total = lax.fori_loop(0, _CW // (_U * _L), inner, total)
        return total

    issue(jnp.int32(0), 0, buf0, sem0)
    issue(jnp.int32(0), 1, buf1, sem1)

    def stripe_body(s, total):
        for cc in range(_NCC):
            buf, sem = bufs[cc % 2], sems[cc % 2]
            wait(buf, sem)
            total = chunk_sum(buf, total)
            if cc + 2 < _NCC:
                issue(s, cc + 2, buf, sem)
            else:
                @pl.when(s + 1 < _NSTRIPE)
                def _():
                    issue(s + 1, cc + 2 - _NCC, buf, sem)
        return total

    total = lax.fori_loop(0, _NSTRIPE, stripe_body,
                          jnp.zeros((_L,), jnp.float32))
    acc_v[...] = total
    pltpu.sync_copy(acc_v, out_hbm.at[pl.ds(wid * _L, _L)])


@functools.partial(
    pl.kernel,
    out_type=jax.ShapeDtypeStruct((_NW * _L,), jnp.float32),
    mesh=plsc.VectorSubcoreMesh(core_axis_name="c", subcore_axis_name="s"),
    scratch_types=[
        pltpu.VMEM((_SR, _CW), jnp.float32),
        pltpu.VMEM((_SR, _CW), jnp.float32),
        pltpu.VMEM((_L,), jnp.float32),
        pltpu.SemaphoreType.DMA,
        pltpu.SemaphoreType.DMA,
    ],
    compiler_params=pltpu.CompilerParams(use_tc_tiling_on_sc=True),
)
def _sc_sum(cos_hbm, out_hbm, buf0, buf1, acc_v, sem0, sem1):
    _sc_body(cos_hbm, out_hbm, buf0, buf1, acc_v, sem0, sem1)


def kernel(cosine, label):
    b, c = cosine.shape
    part = _sc_sum(cosine)
    return (jnp.sum(part) / b).reshape(())


# speedup vs baseline: 1.2231x; 1.2231x over previous
import jax
import jax.numpy as jnp
from jax import lax
from jax.experimental import pallas as pl
from jax.experimental.pallas import tpu as pltpu

_MARGIN = 0.3
_SCALE = 32.0
_BR = 512


def _body(cos_ref, lbl_ref, out_ref):
    br, c = cos_ref.shape
    v = cos_ref[...] * _SCALE
    lbl = lbl_ref[...]
    col = lax.broadcasted_iota(jnp.int32, (br, c), 1)
    onehot = col == lbl
    t = jnp.sum(jnp.where(onehot, v, 0.0), axis=1, keepdims=True) - _SCALE * _MARGIN
    p = jnp.where(onehot, t, jnp.where(v >= t, v, 0.0))
    s = jnp.sum(jnp.exp(p - _SCALE), axis=1, keepdims=True)
    lse = _SCALE + jnp.log(s)
    block_loss = jnp.sum(lse - t, keepdims=True)

    @pl.when(pl.program_id(0) == 0)
    def _():
        out_ref[...] = jnp.zeros_like(out_ref)

    out_ref[...] += block_loss


def kernel(cosine, label):
    b, c = cosine.shape
    grid = b // _BR
    out = pl.pallas_call(
        _body,
        grid=(grid,),
        in_specs=[
            pl.BlockSpec((_BR, c), lambda i: (i, 0)),
            pl.BlockSpec((_BR, 1), lambda i: (i, 0)),
        ],
        out_specs=pl.BlockSpec((1, 1), lambda i: (0, 0)),
        out_shape=jax.ShapeDtypeStruct((1, 1), jnp.float32),
        compiler_params=pltpu.CompilerParams(
            vmem_limit_bytes=110 * 1024 * 1024
        ),
    )(cosine, label.reshape(b, 1))
    return (out[0, 0] / b).reshape(())


# speedup vs baseline: 1.2856x; 1.0510x over previous
import math

import jax
import jax.numpy as jnp
from jax import lax
from jax.experimental import pallas as pl
from jax.experimental.pallas import tpu as pltpu

_MARGIN = 0.3
_SCALE = 32.0
_RB = 128
_NBUF = 6
_K = _SCALE * math.log2(math.e)
_C1 = 1.0 - math.exp(-_SCALE * _MARGIN)


def _body(cos_any, lbl_ref, out_ref, *scratch):
    b, c = cos_any.shape
    nchunk = b // _RB
    bufs = scratch[:_NBUF]
    sems = scratch[_NBUF:]

    def issue(k):
        pltpu.async_copy(
            cos_any.at[pl.ds(k * _RB, _RB)], bufs[k % _NBUF], sems[k % _NBUF]
        )

    def wait(k):
        pltpu.make_async_copy(
            cos_any.at[pl.ds(0, _RB)], bufs[k % _NBUF], sems[k % _NBUF]
        ).wait()

    for k in range(min(_NBUF, nchunk)):
        issue(k)

    loss = jnp.zeros((1, 1), jnp.float32)
    for k in range(nchunk):
        wait(k)
        cos = bufs[k % _NBUF][...]
        lbl = lbl_ref[pl.ds(k * _RB, _RB), :]
        col = lax.broadcasted_iota(jnp.int32, (_RB, c), 1)
        craw = jnp.sum(jnp.where(col == lbl, cos, 0.0), axis=1, keepdims=True)
        thr = craw - _MARGIN
        x = cos * _K - _K
        xm = jnp.where(cos >= thr, x, -_K)
        s_raw = jnp.sum(jnp.exp2(xm), axis=1, keepdims=True)
        s = s_raw - jnp.exp2(craw * _K - _K) * _C1
        lse = _SCALE + jnp.log(s)
        t = _SCALE * thr
        loss = loss + jnp.sum(lse - t, keepdims=True)
        if k + _NBUF < nchunk:
            issue(k + _NBUF)

    out_ref[...] = loss


def kernel(cosine, label):
    b, c = cosine.shape
    out = pl.pallas_call(
        _body,
        in_specs=[
            pl.BlockSpec(memory_space=pl.ANY),
            pl.BlockSpec((b, 1), lambda: (0, 0)),
        ],
        out_specs=pl.BlockSpec((1, 1), lambda: (0, 0)),
        out_shape=jax.ShapeDtypeStruct((1, 1), jnp.float32),
        scratch_shapes=(
            [pltpu.VMEM((_RB, c), jnp.float32) for _ in range(_NBUF)]
            + [pltpu.SemaphoreType.DMA for _ in range(_NBUF)]
        ),
    )(cosine, label.reshape(b, 1))
    return (out[0, 0] / b).reshape(())
